# baseline (speedup 1.0000x reference)
import functools

import numpy as np
import jax
import jax.numpy as jnp
from jax import lax
from jax.experimental import pallas as pl
from jax.experimental.pallas import tpu as pltpu
from jax.experimental.pallas import tpu_sc as plsc

N = 10000
H = 128
NPAD = 12288
NACC = 10240
NRA = NACC // 128
NC, NS = 2, 16
NW = NC * NS
EPT = 10240
ECH = EPT // 128
EPAD = NW * EPT
DUMMY = N
KTOP = 30
FEAT = 385
SORTW = 512
BIG = 2.0
SLA = NACC // NS

_sc_mesh = plsc.VectorSubcoreMesh(core_axis_name="c", subcore_axis_name="s")


@functools.partial(
    pl.kernel,
    out_type=[
        jax.ShapeDtypeStruct((2 * NACC,), jnp.float32),
        jax.ShapeDtypeStruct((2 * NACC,), jnp.float32),
        jax.ShapeDtypeStruct((NPAD, H), jnp.float32),
    ],
    mesh=_sc_mesh,
    scratch_types=[
        pltpu.VMEM((ECH, 128), jnp.int32),
        pltpu.VMEM((ECH, 128), jnp.int32),
        pltpu.VMEM((128,), jnp.int32),
        pltpu.VMEM((128, H), jnp.float32),
        pltpu.VMEM((128,), jnp.float32),
        pltpu.SemaphoreType.DMA,
        pltpu.VMEM_SHARED((NACC,), jnp.float32),
        pltpu.VMEM_SHARED((NACC,), jnp.float32),
    ],
)
def _sc_deg_emb(src_hbm, dst_hbm, z_hbm, ztab_hbm, ones_hbm, zer_hbm,
                outdeg_hbm, indeg_hbm, x_hbm,
                sidx, didx, zidx, zrows, ones_v, sem, sh_out, sh_in):
    c = lax.axis_index("c")
    s = lax.axis_index("s")
    wid = s * NC + c
    off = pl.multiple_of(s * SLA, 128)
    pltpu.sync_copy(zer_hbm, sh_out.at[pl.ds(off, SLA)])
    pltpu.sync_copy(zer_hbm, sh_in.at[pl.ds(off, SLA)])
    pltpu.sync_copy(ones_hbm, ones_v)
    for i in range(3):
        row = wid * 3 + i
        pltpu.sync_copy(z_hbm.at[row], zidx)
        pltpu.async_copy(ztab_hbm.at[zidx], zrows, sem).wait()
        pltpu.sync_copy(zrows, x_hbm.at[pl.ds(pl.multiple_of(row * 128, 128), 128)])
    pltpu.sync_copy(src_hbm.at[wid], sidx)
    pltpu.sync_copy(dst_hbm.at[wid], didx)
    plsc.subcore_barrier()

    def body(j, _):
        pltpu.sync_copy(ones_v, sh_out.at[sidx.at[j]], add=True)
        pltpu.sync_copy(ones_v, sh_in.at[didx.at[j]], add=True)
        return ()

    lax.fori_loop(0, ECH, body, ())
    plsc.subcore_barrier()
    offc = pl.multiple_of(c * NACC + off, 128)
    pltpu.sync_copy(sh_out.at[pl.ds(off, SLA)], outdeg_hbm.at[pl.ds(offc, SLA)])
    pltpu.sync_copy(sh_in.at[pl.ds(off, SLA)], indeg_hbm.at[pl.ds(offc, SLA)])


@functools.partial(
    pl.kernel,
    out_type=jax.ShapeDtypeStruct((2, NACC, H), jnp.float32),
    mesh=_sc_mesh,
    scratch_types=[
        pltpu.VMEM((ECH // 2 + 4, 64), jnp.int32),
        pltpu.VMEM((ECH // 2, 64), jnp.int32),
        pltpu.VMEM((64, H), jnp.float32),
        pltpu.VMEM((64, H), jnp.float32),
        pltpu.VMEM((64, H), jnp.float32),
        pltpu.VMEM((64, H), jnp.float32),
        pltpu.SemaphoreType.DMA,
        pltpu.SemaphoreType.DMA,
        pltpu.SemaphoreType.DMA,
        pltpu.SemaphoreType.DMA,
        pltpu.VMEM_SHARED((NACC, H), jnp.float32),
    ],
)
def _sc_mp(h_hbm, src_hbm, dst_hbm, zer_hbm, zi_hbm, agg_hbm,
           sidx, didx, b0, b1, b2, b3, s0, s1, s2, s3, acc):
    c = lax.axis_index("c")
    s = lax.axis_index("s")
    wid = s * NC + c
    bufs = (b0, b1, b2, b3)
    sems = (s0, s1, s2, s3)
    pltpu.sync_copy(zer_hbm.at[pl.ds(0, 64)], b0)
    pltpu.sync_copy(zer_hbm.at[pl.ds(0, 64)], b1)
    for i in range(SLA // 128):
        off = pl.multiple_of(s * SLA + i * 128, 128)
        pltpu.sync_copy(b0, acc.at[pl.ds(off, 64)])
        pltpu.sync_copy(b1, acc.at[pl.ds(off + 64, 64)])
    plsc.subcore_barrier()
    HC = ECH // 2
    for half in range(4):
        pltpu.sync_copy(src_hbm.at[wid, pl.ds(half * HC, HC)], sidx.at[pl.ds(0, HC)])
        for e in range(4):
            pltpu.sync_copy(zi_hbm, sidx.at[HC + e])
        pltpu.sync_copy(dst_hbm.at[wid, pl.ds(half * HC, HC)], didx)
        for e in range(3):
            pltpu.async_copy(h_hbm.at[sidx.at[e]], bufs[e], sems[e])

        def body(t, _):
            for b in range(4):
                j = t * 4 + b
                nb = (b + 3) % 4
                pltpu.async_copy(h_hbm.at[sidx.at[j + 3]], bufs[nb], sems[nb])
                pltpu.make_async_copy(h_hbm.at[sidx.at[j]], bufs[b], sems[b]).wait()
                pltpu.sync_copy(bufs[b], acc.at[didx.at[j]], add=True)
            return ()

        lax.fori_loop(0, HC // 4, body, ())
        for e in range(3):
            pltpu.make_async_copy(h_hbm.at[sidx.at[HC + e]], bufs[e], sems[e]).wait()
    plsc.subcore_barrier()
    for i in range(SLA // 128):
        off = pl.multiple_of(s * SLA + i * 128, 128)
        pltpu.sync_copy(acc.at[pl.ds(off, 128)], agg_hbm.at[c, pl.ds(off, 128)])


@functools.partial(
    pl.kernel,
    out_type=jax.ShapeDtypeStruct((2 * NACC,), jnp.float32),
    mesh=_sc_mesh,
    scratch_types=[
        pltpu.VMEM((ECH + 2, 128), jnp.int32),
        pltpu.VMEM((ECH, 128), jnp.int32),
        pltpu.VMEM((128,), jnp.float32),
        pltpu.VMEM((128,), jnp.float32),
        pltpu.SemaphoreType.DMA,
        pltpu.SemaphoreType.DMA,
        pltpu.VMEM_SHARED((NACC,), jnp.float32),
    ],
)
def _sc_mp1(h3_hbm, src_hbm, dst_hbm, zer_hbm, zi_hbm, agg_hbm,
            sidx, didx, vals0, vals1, sem0, sem1, shc):
    c = lax.axis_index("c")
    s = lax.axis_index("s")
    wid = s * NC + c
    off = pl.multiple_of(s * SLA, 128)
    pltpu.sync_copy(zer_hbm, shc.at[pl.ds(off, SLA)])
    pltpu.sync_copy(src_hbm.at[wid], sidx.at[pl.ds(0, ECH)])
    pltpu.sync_copy(zi_hbm, sidx.at[ECH])
    pltpu.sync_copy(zi_hbm, sidx.at[ECH + 1])
    pltpu.sync_copy(dst_hbm.at[wid], didx)
    plsc.subcore_barrier()
    pltpu.async_copy(h3_hbm.at[sidx.at[0]], vals0, sem0)

    def body(t, _):
        j0 = t * 2
        pltpu.async_copy(h3_hbm.at[sidx.at[j0 + 1]], vals1, sem1)
        pltpu.make_async_copy(h3_hbm.at[sidx.at[j0]], vals0, sem0).wait()
        pltpu.sync_copy(vals0, shc.at[didx.at[j0]], add=True)
        pltpu.async_copy(h3_hbm.at[sidx.at[j0 + 2]], vals0, sem0)
        pltpu.make_async_copy(h3_hbm.at[sidx.at[j0 + 1]], vals1, sem1).wait()
        pltpu.sync_copy(vals1, shc.at[didx.at[j0 + 1]], add=True)
        return ()

    lax.fori_loop(0, ECH // 2, body, ())
    pltpu.make_async_copy(h3_hbm.at[sidx.at[ECH]], vals0, sem0).wait()
    plsc.subcore_barrier()
    offc = pl.multiple_of(c * NACC + off, 128)
    pltpu.sync_copy(shc.at[pl.ds(off, SLA)], agg_hbm.at[pl.ds(offc, SLA)])


def _t0_body(degs_ref, x_ref, w0_ref, ns_ref, nd_ref, h0_ref):
    degs = degs_ref[...]
    od = degs[:, 0:1] + degs[:, 1:2]
    idg = degs[:, 2:3] + degs[:, 3:4]
    ns = 1.0 / jnp.sqrt(jnp.maximum(od, 1.0))
    nd = 1.0 / jnp.sqrt(jnp.maximum(idg, 1.0))
    ns_ref[...] = ns
    nd_ref[...] = nd
    h0_ref[...] = jnp.dot(x_ref[0:NACC, :] * ns, w0_ref[...],
                          preferred_element_type=jnp.float32)


def _te_body(aggp_ref, nd_ref, ns_ref, b_ref, w_ref, mprev_ref,
             x_ref, m_ref, h_ref):
    a = aggp_ref[...]
    xk = jnp.tanh((a[0] + a[1]) * nd_ref[...] + b_ref[...])
    x_ref[...] = xk
    m_ref[...] = jnp.maximum(mprev_ref[...], jnp.max(xk, axis=1, keepdims=True))
    h_ref[...] = jnp.dot(xk * ns_ref[...], w_ref[...],
                         preferred_element_type=jnp.float32)


def _te3_body(aggp_ref, nd_ref, ns_ref, b_ref, w3_ref, mprev_ref,
              x_ref, m_ref, h3_ref):
    a = aggp_ref[...]
    xk = jnp.tanh((a[0] + a[1]) * nd_ref[...] + b_ref[...])
    x_ref[...] = xk
    m_ref[...] = jnp.maximum(mprev_ref[...], jnp.max(xk, axis=1, keepdims=True))
    h3_ref[...] = jnp.sum((xk * ns_ref[...]) * w3_ref[...], axis=1, keepdims=True)


def _t4_body(agg3m_ref, agg3c_ref, ndm_ref, ndc_ref, b3_ref, m3m_ref,
             x1_ref, x2_ref, x3_ref, pm_ref, wc1_ref, c1b_ref, pe_ref, po_ref,
             w2p_ref, c2b_ref, l3_ref, l1b_ref, l2w_ref, l2b_ref,
             out_ref, sbuf_ref):
    a3 = agg3m_ref[...]
    x4m = jnp.tanh((a3[0] + a3[1]) * ndm_ref[...] + b3_ref[0, 0])
    m = jnp.maximum(m3m_ref[...], x4m)
    gidx = (lax.broadcasted_iota(jnp.int32, (NRA, 128), 0) * 128
            + lax.broadcasted_iota(jnp.int32, (NRA, 128), 1))
    m = jnp.where(gidx < N, m, -BIG)
    sbuf_ref[...] = jnp.full((KTOP + 2, SORTW), BIG, jnp.float32)
    for t in range(KTOP):
        mx = jnp.max(m)
        i = jnp.min(jnp.where(m == mx, gidx, N))
        r1 = x1_ref[pl.ds(i, 1), :]
        r2 = x2_ref[pl.ds(i, 1), :]
        r3 = x3_ref[pl.ds(i, 1), :]
        ac = agg3c_ref[pl.ds(i, 1), :]
        ndl = ndc_ref[pl.ds(i, 1), :]
        x4i = jnp.tanh((ac[:, 0:1] + ac[:, 1:2]) * ndl + b3_ref[...])
        sbuf_ref[pl.ds(t, 1), 0:128] = r1
        sbuf_ref[pl.ds(t, 1), 128:256] = r2
        sbuf_ref[pl.ds(t, 1), 256:384] = r3
        sbuf_ref[pl.ds(t, 1), 384:385] = x4i
        m = jnp.where(gidx == i, -BIG, m)
    x = sbuf_ref[...]
    lane = lax.broadcasted_iota(jnp.int32, (1, SORTW), 1)
    for lk in range(1, 10):
        kk = 1 << lk
        for lj in range(lk - 1, -1, -1):
            j = 1 << lj
            p = jnp.dot(x, pm_ref[lj], preferred_element_type=jnp.float32)
            take_min = ((lane & kk) == 0) == ((lane & j) == 0)
            x = jnp.where(take_min, jnp.minimum(x, p), jnp.maximum(x, p))
    h1 = jnp.maximum(jnp.dot(x[:, 0:FEAT], wc1_ref[...],
                             preferred_element_type=jnp.float32) + c1b_ref[...], 0.0)
    he = jnp.dot(pe_ref[...], h1, preferred_element_type=jnp.float32)
    ho = jnp.dot(po_ref[...], h1, preferred_element_type=jnp.float32)
    h2in = jnp.maximum(he, ho)
    cols = jnp.concatenate([h2in[t:t + 11, :] for t in range(5)], axis=1)
    h2 = jnp.maximum(jnp.dot(cols, w2p_ref[...],
                             preferred_element_type=jnp.float32) + c2b_ref[...], 0.0)
    acc = jnp.zeros((1, 128), jnp.float32)
    for p_ in range(11):
        acc = acc + jnp.dot(h2[p_:p_ + 1, :], l3_ref[p_],
                            preferred_element_type=jnp.float32)
    hl = jnp.maximum(acc + l1b_ref[...], 0.0)
    out_ref[...] = jnp.dot(hl, l2w_ref[...],
                           preferred_element_type=jnp.float32) + l2b_ref[...]


def _np_perm_mats():
    mats = np.zeros((9, SORTW, SORTW), np.float32)
    for lj in range(9):
        j = 1 << lj
        for i in range(SORTW):
            mats[lj, i ^ j, i] = 1.0
    return mats


def _np_pool_mats():
    pe = np.zeros((15, KTOP + 2), np.float32)
    po = np.zeros((15, KTOP + 2), np.float32)
    for p in range(15):
        pe[p, 2 * p] = 1.0
        po[p, 2 * p + 1] = 1.0
    return pe, po


_PM = _np_perm_mats()
_PE, _PO = _np_pool_mats()


def kernel(edge_index, z, z_table, W0, b0, W1, b1, W2, b2, W3, b3,
           conv1_w, conv1_b, conv2_w, conv2_b, lin1_w, lin1_b, lin2_w, lin2_b):
    f32 = jnp.float32
    E = edge_index.shape[1]
    pad_idx = (N + jnp.arange(EPAD - E, dtype=jnp.int32) % (NACC - N)).astype(jnp.int32)
    src = jnp.concatenate([edge_index[0], pad_idx])
    dst = jnp.concatenate([edge_index[1], pad_idx])
    srcp = src.reshape(NW, ECH, 128)
    dstp = dst.reshape(NW, ECH, 128)
    srcp2 = src.reshape(NW, ECH * 2, 64)
    dstp2 = dst.reshape(NW, ECH * 2, 64)
    zi64 = jnp.zeros((64,), jnp.int32)
    zp = jnp.concatenate([z, jnp.zeros((NPAD - N,), jnp.int32)]).reshape(NPAD // 128, 128)
    ones128 = jnp.ones((128,), f32)
    zerSL = jnp.zeros((SLA,), f32)
    zerB = jnp.zeros((128, H), f32)
    zi128 = jnp.zeros((128,), jnp.int32)

    outdeg_p, indeg_p, x = _sc_deg_emb(srcp, dstp, zp, z_table, ones128, zerSL)

    degs = jnp.concatenate([jnp.transpose(outdeg_p.reshape(2, NACC)),
                            jnp.transpose(indeg_p.reshape(2, NACC))], axis=1)
    ns, nd, h0 = pl.pallas_call(
        _t0_body,
        out_shape=[jax.ShapeDtypeStruct((NACC, 1), f32),
                   jax.ShapeDtypeStruct((NACC, 1), f32),
                   jax.ShapeDtypeStruct((NACC, H), f32)],
    )(degs, x, W0)

    te_shapes = [jax.ShapeDtypeStruct((NACC, H), f32),
                 jax.ShapeDtypeStruct((NACC, 1), f32),
                 jax.ShapeDtypeStruct((NACC, H), f32)]
    te3_shapes = [jax.ShapeDtypeStruct((NACC, H), f32),
                  jax.ShapeDtypeStruct((NACC, 1), f32),
                  jax.ShapeDtypeStruct((NACC, 1), f32)]
    mneg = jnp.full((NACC, 1), -BIG, f32)

    aggp0 = _sc_mp(h0, srcp2, dstp2, zerB, zi64)
    x1, m1, h1 = pl.pallas_call(_te_body, out_shape=te_shapes)(
        aggp0, nd, ns, b0.reshape(1, H), W1, mneg)
    aggp1 = _sc_mp(h1, srcp2, dstp2, zerB, zi64)
    x2, m2, h2 = pl.pallas_call(_te_body, out_shape=te_shapes)(
        aggp1, nd, ns, b1.reshape(1, H), W2, m1)
    aggp2 = _sc_mp(h2, srcp2, dstp2, zerB, zi64)
    x3, m3, h3c = pl.pallas_call(_te3_body, out_shape=te3_shapes)(
        aggp2, nd, ns, b2.reshape(1, H), W3.reshape(1, H), m2)

    agg3p = _sc_mp1(h3c.reshape(NACC), srcp, dstp, zerSL, zi128)

    out = pl.pallas_call(
        _t4_body,
        out_shape=jax.ShapeDtypeStruct((1, 1), f32),
        scratch_shapes=[pltpu.VMEM((KTOP + 2, SORTW), f32)],
    )(
        agg3p.reshape(2, NRA, 128),
        jnp.transpose(agg3p.reshape(2, NACC)),
        nd.reshape(NRA, 128),
        nd,
        b3.reshape(1, 1),
        m3.reshape(NRA, 128),
        x1, x2, x3,
        jnp.asarray(_PM),
        jnp.transpose(conv1_w[:, 0, :]),
        conv1_b.reshape(1, 16),
        jnp.asarray(_PE), jnp.asarray(_PO),
        jnp.transpose(conv2_w, (0, 2, 1)).reshape(32, 80).T,
        conv2_b.reshape(1, 32),
        jnp.transpose(lin1_w.reshape(128, 32, 11), (2, 1, 0)),
        lin1_b.reshape(1, 128),
        jnp.transpose(lin2_w),
        lin2_b.reshape(1, 1),
    )
    return out


# speedup vs baseline: 3.8984x; 3.8984x over previous
import functools

import numpy as np
import jax
import jax.numpy as jnp
from jax import lax
from jax.experimental import pallas as pl
from jax.experimental.pallas import tpu as pltpu
from jax.experimental.pallas import tpu_sc as plsc

N = 10000
H = 128
NPAD = 12288
NACC = 10240
NRA = NACC // 128
NC, NS = 2, 16
NW = NC * NS
EPT = 10240
ECH = EPT // 128
EPAD = NW * EPT
DUMMY = N
KTOP = 30
FEAT = 385
SORTW = 512
BIG = 2.0
SLA = NACC // NS

_sc_mesh = plsc.VectorSubcoreMesh(core_axis_name="c", subcore_axis_name="s")


@functools.partial(
    pl.kernel,
    out_type=[
        jax.ShapeDtypeStruct((2 * NACC,), jnp.float32),
        jax.ShapeDtypeStruct((2 * NACC,), jnp.float32),
        jax.ShapeDtypeStruct((NPAD, H), jnp.float32),
    ],
    mesh=_sc_mesh,
    scratch_types=[
        pltpu.VMEM((ECH, 128), jnp.int32),
        pltpu.VMEM((ECH, 128), jnp.int32),
        pltpu.VMEM((128,), jnp.int32),
        pltpu.VMEM((128, H), jnp.float32),
        pltpu.VMEM((128,), jnp.float32),
        pltpu.SemaphoreType.DMA,
        pltpu.VMEM_SHARED((NACC,), jnp.float32),
        pltpu.VMEM_SHARED((NACC,), jnp.float32),
    ],
)
def _sc_deg_emb(src_hbm, dst_hbm, z_hbm, ztab_hbm, ones_hbm, zer_hbm,
                outdeg_hbm, indeg_hbm, x_hbm,
                sidx, didx, zidx, zrows, ones_v, sem, sh_out, sh_in):
    c = lax.axis_index("c")
    s = lax.axis_index("s")
    wid = s * NC + c
    off = pl.multiple_of(s * SLA, 128)
    pltpu.sync_copy(zer_hbm, sh_out.at[pl.ds(off, SLA)])
    pltpu.sync_copy(zer_hbm, sh_in.at[pl.ds(off, SLA)])
    pltpu.sync_copy(ones_hbm, ones_v)
    for i in range(3):
        row = wid * 3 + i
        pltpu.sync_copy(z_hbm.at[row], zidx)
        pltpu.async_copy(ztab_hbm.at[zidx], zrows, sem).wait()
        pltpu.sync_copy(zrows, x_hbm.at[pl.ds(pl.multiple_of(row * 128, 128), 128)])
    pltpu.sync_copy(src_hbm.at[wid], sidx)
    pltpu.sync_copy(dst_hbm.at[wid], didx)
    plsc.subcore_barrier()

    def body(j, _):
        pltpu.sync_copy(ones_v, sh_out.at[sidx.at[j]], add=True)
        pltpu.sync_copy(ones_v, sh_in.at[didx.at[j]], add=True)
        return ()

    lax.fori_loop(0, ECH, body, ())
    plsc.subcore_barrier()
    offc = pl.multiple_of(c * NACC + off, 128)
    pltpu.sync_copy(sh_out.at[pl.ds(off, SLA)], outdeg_hbm.at[pl.ds(offc, SLA)])
    pltpu.sync_copy(sh_in.at[pl.ds(off, SLA)], indeg_hbm.at[pl.ds(offc, SLA)])


@functools.partial(
    pl.kernel,
    out_type=jax.ShapeDtypeStruct((2, NACC, H), jnp.float32),
    mesh=_sc_mesh,
    scratch_types=[
        pltpu.VMEM((ECH, 128), jnp.int32),
        pltpu.VMEM((ECH, 128), jnp.int32),
        pltpu.VMEM((128, H), jnp.float32),
        pltpu.SemaphoreType.DMA,
        pltpu.VMEM_SHARED((NACC, H), jnp.float32),
    ],
)
def _sc_mp(h_hbm, src_hbm, dst_hbm, zer_hbm, zi_hbm, agg_hbm,
           sidx, didx, rows, sem, acc):
    c = lax.axis_index("c")
    s = lax.axis_index("s")
    wid = s * NC + c
    pltpu.sync_copy(zer_hbm, rows)
    for i in range(SLA // 128):
        pltpu.sync_copy(rows, acc.at[pl.ds(pl.multiple_of(s * SLA + i * 128, 128), 128)])
    pltpu.sync_copy(src_hbm.at[wid], sidx)
    pltpu.sync_copy(dst_hbm.at[wid], didx)
    plsc.subcore_barrier()

    def body(j, _):
        pltpu.async_copy(h_hbm.at[sidx.at[j]], rows, sem).wait()
        pltpu.sync_copy(rows, acc.at[didx.at[j]], add=True)
        return ()

    lax.fori_loop(0, ECH, body, ())
    plsc.subcore_barrier()
    for i in range(SLA // 128):
        off = pl.multiple_of(s * SLA + i * 128, 128)
        pltpu.sync_copy(acc.at[pl.ds(off, 128)], agg_hbm.at[c, pl.ds(off, 128)])


@functools.partial(
    pl.kernel,
    out_type=jax.ShapeDtypeStruct((2 * NACC,), jnp.float32),
    mesh=_sc_mesh,
    scratch_types=[
        pltpu.VMEM((ECH, 128), jnp.int32),
        pltpu.VMEM((ECH, 128), jnp.int32),
        pltpu.VMEM((128,), jnp.float32),
        pltpu.SemaphoreType.DMA,
        pltpu.VMEM_SHARED((NACC,), jnp.float32),
    ],
)
def _sc_mp1(h3_hbm, src_hbm, dst_hbm, zer_hbm, zi_hbm, agg_hbm,
            sidx, didx, vals, sem, shc):
    c = lax.axis_index("c")
    s = lax.axis_index("s")
    wid = s * NC + c
    off = pl.multiple_of(s * SLA, 128)
    pltpu.sync_copy(zer_hbm, shc.at[pl.ds(off, SLA)])
    pltpu.sync_copy(src_hbm.at[wid], sidx)
    pltpu.sync_copy(dst_hbm.at[wid], didx)
    plsc.subcore_barrier()

    def body(j, _):
        pltpu.async_copy(h3_hbm.at[sidx.at[j]], vals, sem).wait()
        pltpu.sync_copy(vals, shc.at[didx.at[j]], add=True)
        return ()

    lax.fori_loop(0, ECH, body, ())
    plsc.subcore_barrier()
    offc = pl.multiple_of(c * NACC + off, 128)
    pltpu.sync_copy(shc.at[pl.ds(off, SLA)], agg_hbm.at[pl.ds(offc, SLA)])


def _t0_body(degs_ref, x_ref, w0_ref, ns_ref, nd_ref, h0_ref):
    degs = degs_ref[...]
    od = degs[:, 0:1] + degs[:, 1:2]
    idg = degs[:, 2:3] + degs[:, 3:4]
    ns = 1.0 / jnp.sqrt(jnp.maximum(od, 1.0))
    nd = 1.0 / jnp.sqrt(jnp.maximum(idg, 1.0))
    ns_ref[...] = ns
    nd_ref[...] = nd
    h0_ref[...] = jnp.dot(x_ref[0:NACC, :] * ns, w0_ref[...],
                          preferred_element_type=jnp.float32)


def _te_body(aggp_ref, nd_ref, ns_ref, b_ref, w_ref, mprev_ref,
             x_ref, m_ref, h_ref):
    a = aggp_ref[...]
    xk = jnp.tanh((a[0] + a[1]) * nd_ref[...] + b_ref[...])
    x_ref[...] = xk
    m_ref[...] = jnp.maximum(mprev_ref[...], jnp.max(xk, axis=1, keepdims=True))
    h_ref[...] = jnp.dot(xk * ns_ref[...], w_ref[...],
                         preferred_element_type=jnp.float32)


def _te3_body(aggp_ref, nd_ref, ns_ref, b_ref, w3_ref, mprev_ref,
              x_ref, m_ref, h3_ref):
    a = aggp_ref[...]
    xk = jnp.tanh((a[0] + a[1]) * nd_ref[...] + b_ref[...])
    x_ref[...] = xk
    m_ref[...] = jnp.maximum(mprev_ref[...], jnp.max(xk, axis=1, keepdims=True))
    h3_ref[...] = jnp.sum((xk * ns_ref[...]) * w3_ref[...], axis=1, keepdims=True)


def _t4_body(agg3m_ref, agg3c_ref, ndm_ref, ndc_ref, b3_ref, m3m_ref,
             x1_ref, x2_ref, x3_ref, pm_ref, wc1_ref, c1b_ref, pe_ref, po_ref,
             w2p_ref, c2b_ref, l3_ref, l1b_ref, l2w_ref, l2b_ref,
             out_ref, sbuf_ref):
    a3 = agg3m_ref[...]
    x4m = jnp.tanh((a3[0] + a3[1]) * ndm_ref[...] + b3_ref[0, 0])
    m = jnp.maximum(m3m_ref[...], x4m)
    gidx = (lax.broadcasted_iota(jnp.int32, (NRA, 128), 0) * 128
            + lax.broadcasted_iota(jnp.int32, (NRA, 128), 1))
    m = jnp.where(gidx < N, m, -BIG)
    sbuf_ref[...] = jnp.full((KTOP + 2, SORTW), BIG, jnp.float32)
    for t in range(KTOP):
        mx = jnp.max(m)
        i = jnp.min(jnp.where(m == mx, gidx, N))
        r1 = x1_ref[pl.ds(i, 1), :]
        r2 = x2_ref[pl.ds(i, 1), :]
        r3 = x3_ref[pl.ds(i, 1), :]
        ac = agg3c_ref[pl.ds(i, 1), :]
        ndl = ndc_ref[pl.ds(i, 1), :]
        x4i = jnp.tanh((ac[:, 0:1] + ac[:, 1:2]) * ndl + b3_ref[...])
        sbuf_ref[pl.ds(t, 1), 0:128] = r1
        sbuf_ref[pl.ds(t, 1), 128:256] = r2
        sbuf_ref[pl.ds(t, 1), 256:384] = r3
        sbuf_ref[pl.ds(t, 1), 384:385] = x4i
        m = jnp.where(gidx == i, -BIG, m)
    x = sbuf_ref[...]
    lane = lax.broadcasted_iota(jnp.int32, (1, SORTW), 1)
    for lk in range(1, 10):
        kk = 1 << lk
        for lj in range(lk - 1, -1, -1):
            j = 1 << lj
            p = jnp.dot(x, pm_ref[lj], preferred_element_type=jnp.float32)
            take_min = ((lane & kk) == 0) == ((lane & j) == 0)
            x = jnp.where(take_min, jnp.minimum(x, p), jnp.maximum(x, p))
    h1 = jnp.maximum(jnp.dot(x[:, 0:FEAT], wc1_ref[...],
                             preferred_element_type=jnp.float32) + c1b_ref[...], 0.0)
    he = jnp.dot(pe_ref[...], h1, preferred_element_type=jnp.float32)
    ho = jnp.dot(po_ref[...], h1, preferred_element_type=jnp.float32)
    h2in = jnp.maximum(he, ho)
    cols = jnp.concatenate([h2in[t:t + 11, :] for t in range(5)], axis=1)
    h2 = jnp.maximum(jnp.dot(cols, w2p_ref[...],
                             preferred_element_type=jnp.float32) + c2b_ref[...], 0.0)
    acc = jnp.zeros((1, 128), jnp.float32)
    for p_ in range(11):
        acc = acc + jnp.dot(h2[p_:p_ + 1, :], l3_ref[p_],
                            preferred_element_type=jnp.float32)
    hl = jnp.maximum(acc + l1b_ref[...], 0.0)
    out_ref[...] = jnp.dot(hl, l2w_ref[...],
                           preferred_element_type=jnp.float32) + l2b_ref[...]


def _np_perm_mats():
    mats = np.zeros((9, SORTW, SORTW), np.float32)
    for lj in range(9):
        j = 1 << lj
        for i in range(SORTW):
            mats[lj, i ^ j, i] = 1.0
    return mats


def _np_pool_mats():
    pe = np.zeros((15, KTOP + 2), np.float32)
    po = np.zeros((15, KTOP + 2), np.float32)
    for p in range(15):
        pe[p, 2 * p] = 1.0
        po[p, 2 * p + 1] = 1.0
    return pe, po


_PM = _np_perm_mats()
_PE, _PO = _np_pool_mats()


def kernel(edge_index, z, z_table, W0, b0, W1, b1, W2, b2, W3, b3,
           conv1_w, conv1_b, conv2_w, conv2_b, lin1_w, lin1_b, lin2_w, lin2_b):
    f32 = jnp.float32
    E = edge_index.shape[1]
    pad_idx = (N + jnp.arange(EPAD - E, dtype=jnp.int32) % (NACC - N)).astype(jnp.int32)
    src = jnp.concatenate([edge_index[0], pad_idx])
    dst = jnp.concatenate([edge_index[1], pad_idx])
    srcp = src.reshape(NW, ECH, 128)
    dstp = dst.reshape(NW, ECH, 128)
    zp = jnp.concatenate([z, jnp.zeros((NPAD - N,), jnp.int32)]).reshape(NPAD // 128, 128)
    ones128 = jnp.ones((128,), f32)
    zerSL = jnp.zeros((SLA,), f32)
    zerB = jnp.zeros((128, H), f32)
    zi128 = jnp.zeros((128,), jnp.int32)

    outdeg_p, indeg_p, x = _sc_deg_emb(srcp, dstp, zp, z_table, ones128, zerSL)

    degs = jnp.concatenate([jnp.transpose(outdeg_p.reshape(2, NACC)),
                            jnp.transpose(indeg_p.reshape(2, NACC))], axis=1)
    ns, nd, h0 = pl.pallas_call(
        _t0_body,
        out_shape=[jax.ShapeDtypeStruct((NACC, 1), f32),
                   jax.ShapeDtypeStruct((NACC, 1), f32),
                   jax.ShapeDtypeStruct((NACC, H), f32)],
    )(degs, x, W0)

    te_shapes = [jax.ShapeDtypeStruct((NACC, H), f32),
                 jax.ShapeDtypeStruct((NACC, 1), f32),
                 jax.ShapeDtypeStruct((NACC, H), f32)]
    te3_shapes = [jax.ShapeDtypeStruct((NACC, H), f32),
                  jax.ShapeDtypeStruct((NACC, 1), f32),
                  jax.ShapeDtypeStruct((NACC, 1), f32)]
    mneg = jnp.full((NACC, 1), -BIG, f32)

    aggp0 = _sc_mp(h0, srcp, dstp, zerB, zi128)
    x1, m1, h1 = pl.pallas_call(_te_body, out_shape=te_shapes)(
        aggp0, nd, ns, b0.reshape(1, H), W1, mneg)
    aggp1 = _sc_mp(h1, srcp, dstp, zerB, zi128)
    x2, m2, h2 = pl.pallas_call(_te_body, out_shape=te_shapes)(
        aggp1, nd, ns, b1.reshape(1, H), W2, m1)
    aggp2 = _sc_mp(h2, srcp, dstp, zerB, zi128)
    x3, m3, h3c = pl.pallas_call(_te3_body, out_shape=te3_shapes)(
        aggp2, nd, ns, b2.reshape(1, H), W3.reshape(1, H), m2)

    agg3p = _sc_mp1(h3c.reshape(NACC), srcp, dstp, zerSL, zi128)

    out = pl.pallas_call(
        _t4_body,
        out_shape=jax.ShapeDtypeStruct((1, 1), f32),
        scratch_shapes=[pltpu.VMEM((KTOP + 2, SORTW), f32)],
    )(
        agg3p.reshape(2, NRA, 128),
        jnp.transpose(agg3p.reshape(2, NACC)),
        nd.reshape(NRA, 128),
        nd,
        b3.reshape(1, 1),
        m3.reshape(NRA, 128),
        x1, x2, x3,
        jnp.asarray(_PM),
        jnp.transpose(conv1_w[:, 0, :]),
        conv1_b.reshape(1, 16),
        jnp.asarray(_PE), jnp.asarray(_PO),
        jnp.transpose(conv2_w, (0, 2, 1)).reshape(32, 80).T,
        conv2_b.reshape(1, 32),
        jnp.transpose(lin1_w.reshape(128, 32, 11), (2, 1, 0)),
        lin1_b.reshape(1, 128),
        jnp.transpose(lin2_w),
        lin2_b.reshape(1, 1),
    )
    return out


# speedup vs baseline: 4.6241x; 1.1861x over previous
import functools

import numpy as np
import jax
import jax.numpy as jnp
from jax import lax
from jax.experimental import pallas as pl
from jax.experimental.pallas import tpu as pltpu
from jax.experimental.pallas import tpu_sc as plsc

N = 10000
H = 128
NPAD = 12288
NACC = 10240
NRA = NACC // 128
NC, NS = 2, 16
NW = NC * NS
EPT = 10240
ECH = EPT // 128
EPAD = NW * EPT
DUMMY = N
KTOP = 30
FEAT = 385
SORTW = 512
BIG = 2.0
SLA = NACC // NS

_sc_mesh = plsc.VectorSubcoreMesh(core_axis_name="c", subcore_axis_name="s")


@functools.partial(
    pl.kernel,
    out_type=[
        jax.ShapeDtypeStruct((2 * NACC,), jnp.float32),
        jax.ShapeDtypeStruct((2 * NACC,), jnp.float32),
        jax.ShapeDtypeStruct((NPAD, H), jnp.float32),
    ],
    mesh=_sc_mesh,
    scratch_types=[
        pltpu.VMEM((ECH, 128), jnp.int32),
        pltpu.VMEM((ECH, 128), jnp.int32),
        pltpu.VMEM((128,), jnp.int32),
        pltpu.VMEM((128, H), jnp.float32),
        pltpu.VMEM((128,), jnp.float32),
        pltpu.SemaphoreType.DMA,
        pltpu.VMEM_SHARED((NACC,), jnp.float32),
        pltpu.VMEM_SHARED((NACC,), jnp.float32),
    ],
)
def _sc_deg_emb(src_hbm, dst_hbm, z_hbm, ztab_hbm, ones_hbm, zer_hbm,
                outdeg_hbm, indeg_hbm, x_hbm,
                sidx, didx, zidx, zrows, ones_v, sem, sh_out, sh_in):
    c = lax.axis_index("c")
    s = lax.axis_index("s")
    wid = s * NC + c
    off = pl.multiple_of(s * SLA, 128)
    pltpu.sync_copy(zer_hbm, sh_out.at[pl.ds(off, SLA)])
    pltpu.sync_copy(zer_hbm, sh_in.at[pl.ds(off, SLA)])
    pltpu.sync_copy(ones_hbm, ones_v)
    for i in range(3):
        row = wid * 3 + i
        pltpu.sync_copy(z_hbm.at[row], zidx)
        pltpu.async_copy(ztab_hbm.at[zidx], zrows, sem).wait()
        pltpu.sync_copy(zrows, x_hbm.at[pl.ds(pl.multiple_of(row * 128, 128), 128)])
    pltpu.sync_copy(src_hbm.at[wid], sidx)
    pltpu.sync_copy(dst_hbm.at[wid], didx)
    plsc.subcore_barrier()

    def body(j, _):
        pltpu.sync_copy(ones_v, sh_out.at[sidx.at[j]], add=True)
        pltpu.sync_copy(ones_v, sh_in.at[didx.at[j]], add=True)
        return ()

    lax.fori_loop(0, ECH, body, ())
    plsc.subcore_barrier()
    offc = pl.multiple_of(c * NACC + off, 128)
    pltpu.sync_copy(sh_out.at[pl.ds(off, SLA)], outdeg_hbm.at[pl.ds(offc, SLA)])
    pltpu.sync_copy(sh_in.at[pl.ds(off, SLA)], indeg_hbm.at[pl.ds(offc, SLA)])


@functools.partial(
    pl.kernel,
    out_type=jax.ShapeDtypeStruct((2, NACC, H), jnp.float32),
    mesh=_sc_mesh,
    scratch_types=[
        pltpu.VMEM((ECH // 2, 128), jnp.int32),
        pltpu.VMEM((ECH // 2, 128), jnp.int32),
        pltpu.VMEM((128, H), jnp.float32),
        pltpu.VMEM((128, H), jnp.float32),
        pltpu.SemaphoreType.DMA,
        pltpu.SemaphoreType.DMA,
        pltpu.VMEM_SHARED((NACC, H), jnp.float32),
    ],
)
def _sc_mp(h_hbm, src_hbm, dst_hbm, zer_hbm, zi_hbm, agg_hbm,
           sidx, didx, rows0, rows1, semg, sems, acc):
    c = lax.axis_index("c")
    s = lax.axis_index("s")
    wid = s * NC + c
    pltpu.sync_copy(zer_hbm, rows0)
    for i in range(SLA // 128):
        pltpu.sync_copy(rows0, acc.at[pl.ds(pl.multiple_of(s * SLA + i * 128, 128), 128)])
    plsc.subcore_barrier()
    HC = ECH // 2
    for half in range(2):
        pltpu.sync_copy(src_hbm.at[wid, pl.ds(half * HC, HC)], sidx)
        pltpu.sync_copy(dst_hbm.at[wid, pl.ds(half * HC, HC)], didx)
        pltpu.async_copy(h_hbm.at[sidx.at[0]], rows0, semg).wait()
        pltpu.async_copy(rows0, acc.at[didx.at[0]], sems, add=True)
        pltpu.async_copy(h_hbm.at[sidx.at[1]], rows1, semg).wait()
        pltpu.async_copy(rows1, acc.at[didx.at[1]], sems, add=True)

        def body(t, _):
            for b, rr in ((0, rows0), (1, rows1)):
                j = t * 2 + 2 + b
                pltpu.make_async_copy(h_hbm.at[pl.ds(0, 128)], rr, sems).wait()
                pltpu.async_copy(h_hbm.at[sidx.at[j]], rr, semg).wait()
                pltpu.async_copy(rr, acc.at[didx.at[j]], sems, add=True)
            return ()

        lax.fori_loop(0, (HC - 2) // 2, body, ())
        pltpu.make_async_copy(h_hbm.at[pl.ds(0, 128)], rows0, sems).wait()
        pltpu.make_async_copy(h_hbm.at[pl.ds(0, 128)], rows1, sems).wait()
    plsc.subcore_barrier()
    for i in range(SLA // 128):
        off = pl.multiple_of(s * SLA + i * 128, 128)
        pltpu.sync_copy(acc.at[pl.ds(off, 128)], agg_hbm.at[c, pl.ds(off, 128)])


@functools.partial(
    pl.kernel,
    out_type=jax.ShapeDtypeStruct((2 * NACC,), jnp.float32),
    mesh=_sc_mesh,
    scratch_types=[
        pltpu.VMEM((ECH, 128), jnp.int32),
        pltpu.VMEM((ECH, 128), jnp.int32),
        pltpu.VMEM((128,), jnp.float32),
        pltpu.SemaphoreType.DMA,
        pltpu.VMEM_SHARED((NACC,), jnp.float32),
    ],
)
def _sc_mp1(h3_hbm, src_hbm, dst_hbm, zer_hbm, zi_hbm, agg_hbm,
            sidx, didx, vals, sem, shc):
    c = lax.axis_index("c")
    s = lax.axis_index("s")
    wid = s * NC + c
    off = pl.multiple_of(s * SLA, 128)
    pltpu.sync_copy(zer_hbm, shc.at[pl.ds(off, SLA)])
    pltpu.sync_copy(src_hbm.at[wid], sidx)
    pltpu.sync_copy(dst_hbm.at[wid], didx)
    plsc.subcore_barrier()

    def body(j, _):
        pltpu.async_copy(h3_hbm.at[sidx.at[j]], vals, sem).wait()
        pltpu.sync_copy(vals, shc.at[didx.at[j]], add=True)
        return ()

    lax.fori_loop(0, ECH, body, ())
    plsc.subcore_barrier()
    offc = pl.multiple_of(c * NACC + off, 128)
    pltpu.sync_copy(shc.at[pl.ds(off, SLA)], agg_hbm.at[pl.ds(offc, SLA)])


def _t0_body(degs_ref, x_ref, w0_ref, ns_ref, nd_ref, h0_ref):
    degs = degs_ref[...]
    od = degs[:, 0:1] + degs[:, 1:2]
    idg = degs[:, 2:3] + degs[:, 3:4]
    ns = 1.0 / jnp.sqrt(jnp.maximum(od, 1.0))
    nd = 1.0 / jnp.sqrt(jnp.maximum(idg, 1.0))
    ns_ref[...] = ns
    nd_ref[...] = nd
    h0_ref[...] = jnp.dot(x_ref[0:NACC, :] * ns, w0_ref[...],
                          preferred_element_type=jnp.float32)


def _te_body(aggp_ref, nd_ref, ns_ref, b_ref, w_ref, mprev_ref,
             x_ref, m_ref, h_ref):
    a = aggp_ref[...]
    xk = jnp.tanh((a[0] + a[1]) * nd_ref[...] + b_ref[...])
    x_ref[...] = xk
    m_ref[...] = jnp.maximum(mprev_ref[...], jnp.max(xk, axis=1, keepdims=True))
    h_ref[...] = jnp.dot(xk * ns_ref[...], w_ref[...],
                         preferred_element_type=jnp.float32)


def _te3_body(aggp_ref, nd_ref, ns_ref, b_ref, w3_ref, mprev_ref,
              x_ref, m_ref, h3_ref):
    a = aggp_ref[...]
    xk = jnp.tanh((a[0] + a[1]) * nd_ref[...] + b_ref[...])
    x_ref[...] = xk
    m_ref[...] = jnp.maximum(mprev_ref[...], jnp.max(xk, axis=1, keepdims=True))
    h3_ref[...] = jnp.sum((xk * ns_ref[...]) * w3_ref[...], axis=1, keepdims=True)


def _t4_body(agg3m_ref, agg3c_ref, ndm_ref, ndc_ref, b3_ref, m3m_ref,
             x1_ref, x2_ref, x3_ref, pm_ref, wc1_ref, c1b_ref, pe_ref, po_ref,
             w2p_ref, c2b_ref, l3_ref, l1b_ref, l2w_ref, l2b_ref,
             out_ref, sbuf_ref):
    a3 = agg3m_ref[...]
    x4m = jnp.tanh((a3[0] + a3[1]) * ndm_ref[...] + b3_ref[0, 0])
    m = jnp.maximum(m3m_ref[...], x4m)
    gidx = (lax.broadcasted_iota(jnp.int32, (NRA, 128), 0) * 128
            + lax.broadcasted_iota(jnp.int32, (NRA, 128), 1))
    m = jnp.where(gidx < N, m, -BIG)
    sbuf_ref[...] = jnp.full((KTOP + 2, SORTW), BIG, jnp.float32)
    for t in range(KTOP):
        mx = jnp.max(m)
        i = jnp.min(jnp.where(m == mx, gidx, N))
        r1 = x1_ref[pl.ds(i, 1), :]
        r2 = x2_ref[pl.ds(i, 1), :]
        r3 = x3_ref[pl.ds(i, 1), :]
        ac = agg3c_ref[pl.ds(i, 1), :]
        ndl = ndc_ref[pl.ds(i, 1), :]
        x4i = jnp.tanh((ac[:, 0:1] + ac[:, 1:2]) * ndl + b3_ref[...])
        sbuf_ref[pl.ds(t, 1), 0:128] = r1
        sbuf_ref[pl.ds(t, 1), 128:256] = r2
        sbuf_ref[pl.ds(t, 1), 256:384] = r3
        sbuf_ref[pl.ds(t, 1), 384:385] = x4i
        m = jnp.where(gidx == i, -BIG, m)
    x = sbuf_ref[...]
    lane = lax.broadcasted_iota(jnp.int32, (1, SORTW), 1)
    for lk in range(1, 10):
        kk = 1 << lk
        for lj in range(lk - 1, -1, -1):
            j = 1 << lj
            p = jnp.dot(x, pm_ref[lj], preferred_element_type=jnp.float32)
            take_min = ((lane & kk) == 0) == ((lane & j) == 0)
            x = jnp.where(take_min, jnp.minimum(x, p), jnp.maximum(x, p))
    h1 = jnp.maximum(jnp.dot(x[:, 0:FEAT], wc1_ref[...],
                             preferred_element_type=jnp.float32) + c1b_ref[...], 0.0)
    he = jnp.dot(pe_ref[...], h1, preferred_element_type=jnp.float32)
    ho = jnp.dot(po_ref[...], h1, preferred_element_type=jnp.float32)
    h2in = jnp.maximum(he, ho)
    cols = jnp.concatenate([h2in[t:t + 11, :] for t in range(5)], axis=1)
    h2 = jnp.maximum(jnp.dot(cols, w2p_ref[...],
                             preferred_element_type=jnp.float32) + c2b_ref[...], 0.0)
    acc = jnp.zeros((1, 128), jnp.float32)
    for p_ in range(11):
        acc = acc + jnp.dot(h2[p_:p_ + 1, :], l3_ref[p_],
                            preferred_element_type=jnp.float32)
    hl = jnp.maximum(acc + l1b_ref[...], 0.0)
    out_ref[...] = jnp.dot(hl, l2w_ref[...],
                           preferred_element_type=jnp.float32) + l2b_ref[...]


def _np_perm_mats():
    mats = np.zeros((9, SORTW, SORTW), np.float32)
    for lj in range(9):
        j = 1 << lj
        for i in range(SORTW):
            mats[lj, i ^ j, i] = 1.0
    return mats


def _np_pool_mats():
    pe = np.zeros((15, KTOP + 2), np.float32)
    po = np.zeros((15, KTOP + 2), np.float32)
    for p in range(15):
        pe[p, 2 * p] = 1.0
        po[p, 2 * p + 1] = 1.0
    return pe, po


_PM = _np_perm_mats()
_PE, _PO = _np_pool_mats()


def kernel(edge_index, z, z_table, W0, b0, W1, b1, W2, b2, W3, b3,
           conv1_w, conv1_b, conv2_w, conv2_b, lin1_w, lin1_b, lin2_w, lin2_b):
    f32 = jnp.float32
    E = edge_index.shape[1]
    pad_idx = (N + jnp.arange(EPAD - E, dtype=jnp.int32) % (NACC - N)).astype(jnp.int32)
    src = jnp.concatenate([edge_index[0], pad_idx])
    dst = jnp.concatenate([edge_index[1], pad_idx])
    srcp = src.reshape(NW, ECH, 128)
    dstp = dst.reshape(NW, ECH, 128)
    zp = jnp.concatenate([z, jnp.zeros((NPAD - N,), jnp.int32)]).reshape(NPAD // 128, 128)
    ones128 = jnp.ones((128,), f32)
    zerSL = jnp.zeros((SLA,), f32)
    zerB = jnp.zeros((128, H), f32)
    zi128 = jnp.zeros((128,), jnp.int32)

    outdeg_p, indeg_p, x = _sc_deg_emb(srcp, dstp, zp, z_table, ones128, zerSL)

    degs = jnp.concatenate([jnp.transpose(outdeg_p.reshape(2, NACC)),
                            jnp.transpose(indeg_p.reshape(2, NACC))], axis=1)
    ns, nd, h0 = pl.pallas_call(
        _t0_body,
        out_shape=[jax.ShapeDtypeStruct((NACC, 1), f32),
                   jax.ShapeDtypeStruct((NACC, 1), f32),
                   jax.ShapeDtypeStruct((NACC, H), f32)],
    )(degs, x, W0)

    te_shapes = [jax.ShapeDtypeStruct((NACC, H), f32),
                 jax.ShapeDtypeStruct((NACC, 1), f32),
                 jax.ShapeDtypeStruct((NACC, H), f32)]
    te3_shapes = [jax.ShapeDtypeStruct((NACC, H), f32),
                  jax.ShapeDtypeStruct((NACC, 1), f32),
                  jax.ShapeDtypeStruct((NACC, 1), f32)]
    mneg = jnp.full((NACC, 1), -BIG, f32)

    aggp0 = _sc_mp(h0, srcp, dstp, zerB, zi128)
    x1, m1, h1 = pl.pallas_call(_te_body, out_shape=te_shapes)(
        aggp0, nd, ns, b0.reshape(1, H), W1, mneg)
    aggp1 = _sc_mp(h1, srcp, dstp, zerB, zi128)
    x2, m2, h2 = pl.pallas_call(_te_body, out_shape=te_shapes)(
        aggp1, nd, ns, b1.reshape(1, H), W2, m1)
    aggp2 = _sc_mp(h2, srcp, dstp, zerB, zi128)
    x3, m3, h3c = pl.pallas_call(_te3_body, out_shape=te3_shapes)(
        aggp2, nd, ns, b2.reshape(1, H), W3.reshape(1, H), m2)

    agg3p = _sc_mp1(h3c.reshape(NACC), srcp, dstp, zerSL, zi128)

    out = pl.pallas_call(
        _t4_body,
        out_shape=jax.ShapeDtypeStruct((1, 1), f32),
        scratch_shapes=[pltpu.VMEM((KTOP + 2, SORTW), f32)],
    )(
        agg3p.reshape(2, NRA, 128),
        jnp.transpose(agg3p.reshape(2, NACC)),
        nd.reshape(NRA, 128),
        nd,
        b3.reshape(1, 1),
        m3.reshape(NRA, 128),
        x1, x2, x3,
        jnp.asarray(_PM),
        jnp.transpose(conv1_w[:, 0, :]),
        conv1_b.reshape(1, 16),
        jnp.asarray(_PE), jnp.asarray(_PO),
        jnp.transpose(conv2_w, (0, 2, 1)).reshape(32, 80).T,
        conv2_b.reshape(1, 32),
        jnp.transpose(lin1_w.reshape(128, 32, 11), (2, 1, 0)),
        lin1_b.reshape(1, 128),
        jnp.transpose(lin2_w),
        lin2_b.reshape(1, 1),
    )
    return out
